# baseline (speedup 1.0000x reference)
import jax
import jax.numpy as jnp
from jax.experimental import pallas as pl
from jax.experimental.pallas import tpu as pltpu

B, S, D, E = 4, 8192, 2048, 64
CHUNK = 512
GRID = S // CHUNK


def _body(h_ref, w_ref, o_ref, acc_ref):
    i = pl.program_id(0)

    @pl.when(i == 0)
    def _init():
        acc_ref[...] = jnp.zeros_like(acc_ref)

    acc_ref[...] += jnp.sum(h_ref[...], axis=1)

    @pl.when(i == GRID - 1)
    def _fin():
        pooled = acc_ref[...] * (1.0 / S)
        logits = jax.lax.dot_general(
            pooled, w_ref[...],
            dimension_numbers=(((1,), (1,)), ((), ())),
            preferred_element_type=jnp.float32,
        )
        m = jnp.max(logits, axis=-1, keepdims=True)
        e = jnp.exp(logits - m)
        o_ref[...] = e / jnp.sum(e, axis=-1, keepdims=True)


def kernel(hidden_states, W):
    return pl.pallas_call(
        _body,
        grid=(GRID,),
        in_specs=[
            pl.BlockSpec((B, CHUNK, D), lambda i: (0, i, 0)),
            pl.BlockSpec((E, D), lambda i: (0, 0)),
        ],
        out_specs=pl.BlockSpec((B, E), lambda i: (0, 0)),
        out_shape=jax.ShapeDtypeStruct((B, E), jnp.float32),
        scratch_shapes=[pltpu.VMEM((B, D), jnp.float32)],
    )(hidden_states, W)
